# Initial kernel scaffold; baseline (speedup 1.0000x reference)
import jax
import jax.numpy as jnp
from jax.experimental import pallas as pl


def kernel(v_types, v_paths, adj, v_sizes, type_table, path_table, Ws1, bs1, Ws2, bs2, Wh, bh, eps, Wg1, bg1, Wg2, bg2, Wp1, bp1, Wp2, bp2, Wgp, bgp):
    raise NotImplementedError("write your pallas kernel here")


# speedup vs baseline: 3.6708x; 3.6708x over previous
import functools

import jax
import jax.numpy as jnp
from jax.experimental import pallas as pl
from jax.experimental.pallas import tpu as pltpu

_B = 4096
_MAXN = 64
_HID = 64


def _digin_kernel(adjT_ref, vtT_ref, vpT_ref, vsT_ref, ttT_ref, ptT_ref,
                  WhtT_ref, WhpT_ref, bhc_ref, eps_ref,
                  Wg1T_ref, bg1c_ref, Wg2T_ref, bg2c_ref,
                  Wp1c_ref, bp1c_ref, Wp2T_ref, bp2c_ref,
                  Ws1T_ref, bs1c_ref, Ws2T_ref, bs2c_ref,
                  WgpgT_ref, WgpsT_ref, bgpc_ref,
                  outT_ref, h_ref, q_ref, *, bb):
    v = pl.program_id(1)

    @pl.when(v == 0)
    def _init():
        h_ref[...] = jnp.zeros_like(h_ref)
        q_ref[...] = jnp.broadcast_to(bp1c_ref[...], q_ref.shape)

    vt = vtT_ref[0]
    vp = vpT_ref[0]
    oh_t = (jax.lax.broadcasted_iota(jnp.int32, (32, bb), 0) == vt
            ).astype(jnp.float32)
    oh_p = (jax.lax.broadcasted_iota(jnp.int32, (8, bb), 0) == vp
            ).astype(jnp.float32)
    TtT = WhtT_ref[...] @ ttT_ref[...]
    PtT = WhpT_ref[...] @ ptT_ref[...]
    hv_t = TtT @ oh_t + PtT @ oh_p + bhc_ref[...]

    m = adjT_ref[0].astype(jnp.float32)
    u_iota = jax.lax.broadcasted_iota(jnp.int32, (_MAXN, bb), 0)
    m = jnp.where(u_iota < v, m, 0.0)
    h = h_ref[...]
    nsum_t = jnp.sum(m[:, None, :] * h, axis=0)

    x_t = hv_t + eps_ref[...] * hv_t + nsum_t
    a = jnp.maximum(Wg1T_ref[...] @ x_t + bg1c_ref[...], 0.0)
    hnew_t = Wg2T_ref[...] @ a + bg2c_ref[...]
    h_ref[v] = hnew_t

    q_ref[...] += Wp1c_ref[0] @ hnew_t

    @pl.when(v == _MAXN - 1)
    def _readout():
        q = jnp.maximum(q_ref[...], 0.0)
        g_t = Wp2T_ref[...] @ q + bp2c_ref[...]
        sa = jnp.maximum(Ws1T_ref[...] @ vsT_ref[...] + bs1c_ref[...], 0.0)
        s_t = Ws2T_ref[...] @ sa + bs2c_ref[...]
        outT_ref[...] = WgpgT_ref[...] @ g_t + WgpsT_ref[...] @ s_t + bgpc_ref[...]


def kernel(v_types, v_paths, adj, v_sizes, type_table, path_table, Ws1, bs1,
           Ws2, bs2, Wh, bh, eps, Wg1, bg1, Wg2, bg2, Wp1, bp1, Wp2, bp2,
           Wgp, bgp):
    BB = 512
    NB = _B // BB

    adjT = jnp.transpose(adj, (1, 2, 0))
    vtT = jnp.transpose(v_types, (1, 0)).reshape(_MAXN, 1, _B)
    vpT = jnp.transpose(v_paths, (1, 0)).reshape(_MAXN, 1, _B)
    vsT = jnp.transpose(v_sizes, (1, 0))
    Wp1c = jnp.transpose(Wp1.reshape(_MAXN, _HID, 4 * _HID), (0, 2, 1))

    col = lambda x: x.reshape(-1, 1)
    rep = lambda shape: pl.BlockSpec(shape, lambda i, v: (0,) * len(shape))

    outT = pl.pallas_call(
        functools.partial(_digin_kernel, bb=BB),
        grid=(NB, _MAXN),
        in_specs=[
            pl.BlockSpec((1, _MAXN, BB), lambda i, v: (v, 0, i)),
            pl.BlockSpec((1, 1, BB), lambda i, v: (v, 0, i)),
            pl.BlockSpec((1, 1, BB), lambda i, v: (v, 0, i)),
            pl.BlockSpec((3 * _MAXN, BB), lambda i, v: (0, i)),
            rep((16, 32)),
            rep((16, 8)),
            rep((_HID, 16)),
            rep((_HID, 16)),
            rep((_HID, 1)),
            rep((1, 1)),
            rep((_HID, _HID)),
            rep((_HID, 1)),
            rep((_HID, _HID)),
            rep((_HID, 1)),
            pl.BlockSpec((1, 4 * _HID, _HID), lambda i, v: (v, 0, 0)),
            rep((4 * _HID, 1)),
            rep((_HID, 4 * _HID)),
            rep((_HID, 1)),
            rep((16, 3 * _MAXN)),
            rep((16, 1)),
            rep((8, 16)),
            rep((8, 1)),
            rep((_HID, _HID)),
            rep((_HID, 8)),
            rep((_HID, 1)),
        ],
        out_specs=pl.BlockSpec((_HID, BB), lambda i, v: (0, i)),
        out_shape=jax.ShapeDtypeStruct((_HID, _B), jnp.float32),
        scratch_shapes=[
            pltpu.VMEM((_MAXN, _HID, BB), jnp.float32),
            pltpu.VMEM((4 * _HID, BB), jnp.float32),
        ],
    )(adjT, vtT, vpT, vsT,
      type_table.T, path_table.T, Wh[:16].T, Wh[16:].T, col(bh),
      eps.reshape(1, 1),
      Wg1.T, col(bg1), Wg2.T, col(bg2),
      Wp1c, col(bp1), Wp2.T, col(bp2),
      Ws1.T, col(bs1), Ws2.T, col(bs2),
      Wgp[:_HID].T, Wgp[_HID:].T, col(bgp))
    return outT.T


# speedup vs baseline: 4.0715x; 1.1092x over previous
import functools

import jax
import jax.numpy as jnp
from jax.experimental import pallas as pl
from jax.experimental.pallas import tpu as pltpu

_B = 4096
_MAXN = 64
_HID = 64


def _digin_kernel(adjC_ref, vtT_ref, vpT_ref, vsT_ref, ttT_ref, ptT_ref,
                  WhtT_ref, WhpT_ref, bhc_ref, eps_ref,
                  Wg1T_ref, bg1c_ref, Wg2T_ref, bg2c_ref,
                  Wp1c_ref, bp1c_ref, Wp2T_ref, bp2c_ref,
                  Ws1T_ref, bs1c_ref, Ws2T_ref, bs2c_ref,
                  WgpgT_ref, WgpsT_ref, bgpc_ref,
                  outT_ref, h_ref, q_ref, *, bb):
    v = pl.program_id(1)

    @pl.when(v == 0)
    def _init():
        h_ref[...] = jnp.zeros_like(h_ref)
        q_ref[...] = jnp.broadcast_to(bp1c_ref[...], q_ref.shape)

    vt = vtT_ref[0]
    vp = vpT_ref[0]
    oh_t = (jax.lax.broadcasted_iota(jnp.int32, (32, bb), 0) == vt
            ).astype(jnp.float32)
    oh_p = (jax.lax.broadcasted_iota(jnp.int32, (8, bb), 0) == vp
            ).astype(jnp.float32)
    TtT = WhtT_ref[...] @ ttT_ref[...]
    PtT = WhpT_ref[...] @ ptT_ref[...]
    hv_t = TtT @ oh_t + PtT @ oh_p + bhc_ref[...]

    def _chunk(c, acc):
        mc = adjC_ref[c, 0].astype(jnp.float32)
        u8 = jax.lax.broadcasted_iota(jnp.int32, (8, bb), 0) + c * 8
        mc = jnp.where(u8 < v, mc, 0.0)
        hc = h_ref[pl.ds(c * 8, 8)]
        return acc + jnp.sum(mc[:, None, :] * hc, axis=0)

    nsum_t = jax.lax.fori_loop(0, (v >> 3) + 1, _chunk,
                               jnp.zeros((_HID, bb), jnp.float32))

    x_t = hv_t + eps_ref[...] * hv_t + nsum_t
    a = jnp.maximum(Wg1T_ref[...] @ x_t + bg1c_ref[...], 0.0)
    hnew_t = Wg2T_ref[...] @ a + bg2c_ref[...]
    h_ref[v] = hnew_t

    q_ref[...] += Wp1c_ref[0] @ hnew_t

    @pl.when(v == _MAXN - 1)
    def _readout():
        q = jnp.maximum(q_ref[...], 0.0)
        g_t = Wp2T_ref[...] @ q + bp2c_ref[...]
        sa = jnp.maximum(Ws1T_ref[...] @ vsT_ref[...] + bs1c_ref[...], 0.0)
        s_t = Ws2T_ref[...] @ sa + bs2c_ref[...]
        outT_ref[...] = WgpgT_ref[...] @ g_t + WgpsT_ref[...] @ s_t + bgpc_ref[...]


def kernel(v_types, v_paths, adj, v_sizes, type_table, path_table, Ws1, bs1,
           Ws2, bs2, Wh, bh, eps, Wg1, bg1, Wg2, bg2, Wp1, bp1, Wp2, bp2,
           Wgp, bgp):
    BB = 512
    NB = _B // BB

    adjC = jnp.transpose(jnp.transpose(adj, (1, 2, 0)).reshape(_MAXN, 8, 8, _B),
                         (1, 0, 2, 3))
    vtT = jnp.transpose(v_types, (1, 0)).reshape(_MAXN, 1, _B)
    vpT = jnp.transpose(v_paths, (1, 0)).reshape(_MAXN, 1, _B)
    vsT = jnp.transpose(v_sizes, (1, 0))
    Wp1c = jnp.transpose(Wp1.reshape(_MAXN, _HID, 4 * _HID), (0, 2, 1))

    col = lambda x: x.reshape(-1, 1)
    rep = lambda shape: pl.BlockSpec(shape, lambda i, v: (0,) * len(shape))

    outT = pl.pallas_call(
        functools.partial(_digin_kernel, bb=BB),
        grid=(NB, _MAXN),
        in_specs=[
            pl.BlockSpec((8, 1, 8, BB), lambda i, v: (0, v, 0, i)),
            pl.BlockSpec((1, 1, BB), lambda i, v: (v, 0, i)),
            pl.BlockSpec((1, 1, BB), lambda i, v: (v, 0, i)),
            pl.BlockSpec((3 * _MAXN, BB), lambda i, v: (0, i)),
            rep((16, 32)),
            rep((16, 8)),
            rep((_HID, 16)),
            rep((_HID, 16)),
            rep((_HID, 1)),
            rep((1, 1)),
            rep((_HID, _HID)),
            rep((_HID, 1)),
            rep((_HID, _HID)),
            rep((_HID, 1)),
            pl.BlockSpec((1, 4 * _HID, _HID), lambda i, v: (v, 0, 0)),
            rep((4 * _HID, 1)),
            rep((_HID, 4 * _HID)),
            rep((_HID, 1)),
            rep((16, 3 * _MAXN)),
            rep((16, 1)),
            rep((8, 16)),
            rep((8, 1)),
            rep((_HID, _HID)),
            rep((_HID, 8)),
            rep((_HID, 1)),
        ],
        out_specs=pl.BlockSpec((_HID, BB), lambda i, v: (0, i)),
        out_shape=jax.ShapeDtypeStruct((_HID, _B), jnp.float32),
        scratch_shapes=[
            pltpu.VMEM((_MAXN, _HID, BB), jnp.float32),
            pltpu.VMEM((4 * _HID, BB), jnp.float32),
        ],
    )(adjC, vtT, vpT, vsT,
      type_table.T, path_table.T, Wh[:16].T, Wh[16:].T, col(bh),
      eps.reshape(1, 1),
      Wg1.T, col(bg1), Wg2.T, col(bg2),
      Wp1c, col(bp1), Wp2.T, col(bp2),
      Ws1.T, col(bs1), Ws2.T, col(bs2),
      Wgp[:_HID].T, Wgp[_HID:].T, col(bgp))
    return outT.T


# speedup vs baseline: 6.3425x; 1.5578x over previous
import functools

import jax
import jax.numpy as jnp
from jax.experimental import pallas as pl
from jax.experimental.pallas import tpu as pltpu

_B = 4096
_MAXN = 64
_HID = 64
_G = 8
_NG = _MAXN // _G


def _digin_kernel(adjE_ref, vtG_ref, vpG_ref, vsT_ref, ttT_ref, ptT_ref,
                  WhtT_ref, WhpT_ref, bhc_ref, eps_ref,
                  Wg1T_ref, bg1c_ref, Wg2T_ref, bg2c_ref,
                  Wp1g_ref, bp1c_ref, Wp2T_ref, bp2c_ref,
                  Ws1T_ref, bs1c_ref, Ws2T_ref, bs2c_ref,
                  WgpgT_ref, WgpsT_ref, bgpc_ref,
                  outT_ref, h_ref, q_ref, *, bb):
    g = pl.program_id(1)

    @pl.when(g == 0)
    def _init():
        q_ref[...] = jnp.broadcast_to(bp1c_ref[...], q_ref.shape)

    TtT = WhtT_ref[...] @ ttT_ref[...]
    PtT = WhpT_ref[...] @ ptT_ref[...]
    vt8 = vtG_ref[0]
    vp8 = vpG_ref[0]
    iota32 = jax.lax.broadcasted_iota(jnp.int32, (32, bb), 0)
    iota8 = jax.lax.broadcasted_iota(jnp.int32, (8, bb), 0)
    hv = []
    for j in range(_G):
        oh_t = (iota32 == vt8[j:j + 1]).astype(jnp.float32)
        oh_p = (iota8 == vp8[j:j + 1]).astype(jnp.float32)
        hv.append(TtT @ oh_t + PtT @ oh_p + bhc_ref[...])

    def _chunk(c, accs):
        mc = adjE_ref[c].astype(jnp.float32)
        hc = h_ref[pl.ds(c * _G, _G)]
        return tuple(
            accs[j] + jnp.sum(mc[j][:, None, :] * hc, axis=0)
            for j in range(_G))

    zero = jnp.zeros((_HID, bb), jnp.float32)
    accs = jax.lax.fori_loop(0, g, _chunk, (zero,) * _G)

    md = adjE_ref[g].astype(jnp.float32)
    epsv = eps_ref[...]
    hnew = []
    for j in range(_G):
        nsum = accs[j]
        for jp in range(j):
            nsum = nsum + md[j][jp:jp + 1] * hnew[jp]
        x = hv[j] + epsv * hv[j] + nsum
        a = jnp.maximum(Wg1T_ref[...] @ x + bg1c_ref[...], 0.0)
        hnew.append(Wg2T_ref[...] @ a + bg2c_ref[...])

    Hg = jnp.concatenate(hnew, axis=0)
    h_ref[pl.ds(g * _G, _G)] = Hg.reshape(_G, _HID, bb)

    q_ref[...] += Wp1g_ref[0] @ Hg

    @pl.when(g == _NG - 1)
    def _readout():
        q = jnp.maximum(q_ref[...], 0.0)
        g_t = Wp2T_ref[...] @ q + bp2c_ref[...]
        sa = jnp.maximum(Ws1T_ref[...] @ vsT_ref[...] + bs1c_ref[...], 0.0)
        s_t = Ws2T_ref[...] @ sa + bs2c_ref[...]
        outT_ref[...] = WgpgT_ref[...] @ g_t + WgpsT_ref[...] @ s_t + bgpc_ref[...]


def kernel(v_types, v_paths, adj, v_sizes, type_table, path_table, Ws1, bs1,
           Ws2, bs2, Wh, bh, eps, Wg1, bg1, Wg2, bg2, Wp1, bp1, Wp2, bp2,
           Wgp, bgp):
    BB = 512
    NB = _B // BB

    adjE = jnp.transpose(jnp.transpose(adj, (1, 2, 0)).reshape(_MAXN, 8, 8, _B),
                         (1, 0, 2, 3))
    vtG = jnp.transpose(v_types, (1, 0)).reshape(_NG, _G, _B)
    vpG = jnp.transpose(v_paths, (1, 0)).reshape(_NG, _G, _B)
    vsT = jnp.transpose(v_sizes, (1, 0))
    Wp1g = jnp.transpose(Wp1.reshape(_NG, _G * _HID, 4 * _HID), (0, 2, 1))

    col = lambda x: x.reshape(-1, 1)
    rep = lambda shape: pl.BlockSpec(shape, lambda i, g: (0,) * len(shape))

    outT = pl.pallas_call(
        functools.partial(_digin_kernel, bb=BB),
        grid=(NB, _NG),
        in_specs=[
            pl.BlockSpec((8, _G, 8, BB), lambda i, g: (0, g, 0, i)),
            pl.BlockSpec((1, _G, BB), lambda i, g: (g, 0, i)),
            pl.BlockSpec((1, _G, BB), lambda i, g: (g, 0, i)),
            pl.BlockSpec((3 * _MAXN, BB), lambda i, g: (0, i)),
            rep((16, 32)),
            rep((16, 8)),
            rep((_HID, 16)),
            rep((_HID, 16)),
            rep((_HID, 1)),
            rep((1, 1)),
            rep((_HID, _HID)),
            rep((_HID, 1)),
            rep((_HID, _HID)),
            rep((_HID, 1)),
            pl.BlockSpec((1, 4 * _HID, _G * _HID),
                         lambda i, g: (g, 0, 0)),
            rep((4 * _HID, 1)),
            rep((_HID, 4 * _HID)),
            rep((_HID, 1)),
            rep((16, 3 * _MAXN)),
            rep((16, 1)),
            rep((8, 16)),
            rep((8, 1)),
            rep((_HID, _HID)),
            rep((_HID, 8)),
            rep((_HID, 1)),
        ],
        out_specs=pl.BlockSpec((_HID, BB), lambda i, g: (0, i)),
        out_shape=jax.ShapeDtypeStruct((_HID, _B), jnp.float32),
        scratch_shapes=[
            pltpu.VMEM((_MAXN, _HID, BB), jnp.float32),
            pltpu.VMEM((4 * _HID, BB), jnp.float32),
        ],
    )(adjE, vtG, vpG, vsT,
      type_table.T, path_table.T, Wh[:16].T, Wh[16:].T, col(bh),
      eps.reshape(1, 1),
      Wg1.T, col(bg1), Wg2.T, col(bg2),
      Wp1g, col(bp1), Wp2.T, col(bp2),
      Ws1.T, col(bs1), Ws2.T, col(bs2),
      Wgp[:_HID].T, Wgp[_HID:].T, col(bgp))
    return outT.T


# speedup vs baseline: 7.2099x; 1.1368x over previous
import functools

import jax
import jax.numpy as jnp
from jax.experimental import pallas as pl
from jax.experimental.pallas import tpu as pltpu

_B = 4096
_MAXN = 64
_HID = 64
_G = 8
_NG = _MAXN // _G


def _digin_kernel(adjE_ref, vtG_ref, vpG_ref, vsT_ref, ttT_ref, ptT_ref,
                  WhtT_ref, WhpT_ref, bhc_ref, eps_ref,
                  Wg1T_ref, bg1c_ref, Wg2T_ref, bg2c_ref,
                  Wp1g_ref, bp1c_ref, Wp2T_ref, bp2c_ref,
                  Ws1T_ref, bs1c_ref, Ws2T_ref, bs2c_ref,
                  WgpgT_ref, WgpsT_ref, bgpc_ref,
                  outT_ref, h_ref, q_ref, *, bb):
    g = pl.program_id(1)

    @pl.when(g == 0)
    def _init():
        q_ref[...] = jnp.broadcast_to(bp1c_ref[...], q_ref.shape)

    TtT = WhtT_ref[...] @ ttT_ref[...]
    PtT = WhpT_ref[...] @ ptT_ref[...]
    epsp1 = 1.0 + eps_ref[...]
    W40 = jnp.concatenate([TtT, PtT], axis=1) * epsp1
    bhe = bhc_ref[...] * epsp1
    vt8 = vtG_ref[0]
    vp8 = vpG_ref[0]
    iota32 = jax.lax.broadcasted_iota(jnp.int32, (32, bb), 0)
    iota8 = jax.lax.broadcasted_iota(jnp.int32, (8, bb), 0)
    xe = []
    for j in range(_G):
        oh_t = (iota32 == vt8[j:j + 1]).astype(jnp.float32)
        oh_p = (iota8 == vp8[j:j + 1]).astype(jnp.float32)
        oh = jnp.concatenate([oh_t, oh_p], axis=0)
        xe.append(W40 @ oh + bhe)

    def _chunk(c, accs):
        mc = adjE_ref[c]
        hc = h_ref[pl.ds(c * _G, _G)]
        return tuple(
            accs[j] + jnp.sum(mc[j][:, None, :] * hc, axis=0)
            for j in range(_G))

    accs = jax.lax.fori_loop(0, g, _chunk, tuple(xe))

    md = adjE_ref[g]
    hnew = []
    for j in range(_G):
        x = accs[j]
        for jp in range(j):
            x = x + md[j][jp:jp + 1] * hnew[jp]
        a = jnp.maximum(Wg1T_ref[...] @ x + bg1c_ref[...], 0.0)
        hnew.append(Wg2T_ref[...] @ a + bg2c_ref[...])

    Hg = jnp.concatenate(hnew, axis=0)
    h_ref[pl.ds(g * _G, _G)] = Hg.reshape(_G, _HID, bb)

    q_ref[...] += Wp1g_ref[0] @ Hg

    @pl.when(g == _NG - 1)
    def _readout():
        q = jnp.maximum(q_ref[...], 0.0)
        g_t = Wp2T_ref[...] @ q + bp2c_ref[...]
        sa = jnp.maximum(Ws1T_ref[...] @ vsT_ref[...] + bs1c_ref[...], 0.0)
        s_t = Ws2T_ref[...] @ sa + bs2c_ref[...]
        outT_ref[...] = WgpgT_ref[...] @ g_t + WgpsT_ref[...] @ s_t + bgpc_ref[...]


def kernel(v_types, v_paths, adj, v_sizes, type_table, path_table, Ws1, bs1,
           Ws2, bs2, Wh, bh, eps, Wg1, bg1, Wg2, bg2, Wp1, bp1, Wp2, bp2,
           Wgp, bgp):
    BB = 1024
    NB = _B // BB

    adjE = jnp.transpose(jnp.transpose(adj, (1, 2, 0)).reshape(_MAXN, 8, 8, _B),
                         (1, 0, 2, 3)).astype(jnp.float32)
    vtG = jnp.transpose(v_types, (1, 0)).reshape(_NG, _G, _B)
    vpG = jnp.transpose(v_paths, (1, 0)).reshape(_NG, _G, _B)
    vsT = jnp.transpose(v_sizes, (1, 0))
    Wp1g = jnp.transpose(Wp1.reshape(_NG, _G * _HID, 4 * _HID), (0, 2, 1))

    col = lambda x: x.reshape(-1, 1)
    rep = lambda shape: pl.BlockSpec(shape, lambda i, g: (0,) * len(shape))

    outT = pl.pallas_call(
        functools.partial(_digin_kernel, bb=BB),
        grid=(NB, _NG),
        in_specs=[
            pl.BlockSpec((8, _G, 8, BB), lambda i, g: (0, g, 0, i)),
            pl.BlockSpec((1, _G, BB), lambda i, g: (g, 0, i)),
            pl.BlockSpec((1, _G, BB), lambda i, g: (g, 0, i)),
            pl.BlockSpec((3 * _MAXN, BB), lambda i, g: (0, i)),
            rep((16, 32)),
            rep((16, 8)),
            rep((_HID, 16)),
            rep((_HID, 16)),
            rep((_HID, 1)),
            rep((1, 1)),
            rep((_HID, _HID)),
            rep((_HID, 1)),
            rep((_HID, _HID)),
            rep((_HID, 1)),
            pl.BlockSpec((1, 4 * _HID, _G * _HID),
                         lambda i, g: (g, 0, 0)),
            rep((4 * _HID, 1)),
            rep((_HID, 4 * _HID)),
            rep((_HID, 1)),
            rep((16, 3 * _MAXN)),
            rep((16, 1)),
            rep((8, 16)),
            rep((8, 1)),
            rep((_HID, _HID)),
            rep((_HID, 8)),
            rep((_HID, 1)),
        ],
        out_specs=pl.BlockSpec((_HID, BB), lambda i, g: (0, i)),
        out_shape=jax.ShapeDtypeStruct((_HID, _B), jnp.float32),
        scratch_shapes=[
            pltpu.VMEM((_MAXN, _HID, BB), jnp.float32),
            pltpu.VMEM((4 * _HID, BB), jnp.float32),
        ],
    )(adjE, vtG, vpG, vsT,
      type_table.T, path_table.T, Wh[:16].T, Wh[16:].T, col(bh),
      eps.reshape(1, 1),
      Wg1.T, col(bg1), Wg2.T, col(bg2),
      Wp1g, col(bp1), Wp2.T, col(bp2),
      Ws1.T, col(bs1), Ws2.T, col(bs2),
      Wgp[:_HID].T, Wgp[_HID:].T, col(bgp))
    return outT.T


# speedup vs baseline: 8.1582x; 1.1315x over previous
import functools

import jax
import jax.numpy as jnp
from jax.experimental import pallas as pl
from jax.experimental.pallas import tpu as pltpu

_B = 4096
_MAXN = 64
_HID = 64
_G = 8
_NG = _MAXN // _G


def _digin_kernel(adjE_ref, vtG_ref, vpG_ref, vsT_ref, ttT_ref, ptT_ref,
                  WhtT_ref, WhpT_ref, bhc_ref, eps_ref,
                  Wg1T_ref, bg1c_ref, Wg2T_ref, bg2c_ref,
                  Wp1g_ref, bp1c_ref, Wp2T_ref, bp2c_ref,
                  Ws1T_ref, bs1c_ref, Ws2T_ref, bs2c_ref,
                  WgpgT_ref, WgpsT_ref, bgpc_ref,
                  outT_ref, h_ref, q_ref, nsum_ref, *, bb):
    g = pl.program_id(1)

    @pl.when(g == 0)
    def _init():
        q_ref[...] = jnp.broadcast_to(bp1c_ref[...], q_ref.shape)

    TtT = WhtT_ref[...] @ ttT_ref[...]
    PtT = WhpT_ref[...] @ ptT_ref[...]
    epsp1 = 1.0 + eps_ref[...]
    W40 = jnp.concatenate([TtT, PtT], axis=1) * epsp1
    bhe = bhc_ref[...] * epsp1
    vt8 = vtG_ref[0]
    vp8 = vpG_ref[0]
    iota32 = jax.lax.broadcasted_iota(jnp.int32, (32, bb), 0)
    iota8 = jax.lax.broadcasted_iota(jnp.int32, (8, bb), 0)

    nsum_ref[...] = jnp.zeros_like(nsum_ref)
    _HS = 16
    for s in range(_HID // _HS):
        lo = s * _HS

        def _chunk(c, carry, lo=lo):
            mc = adjE_ref[c]
            hc = h_ref[pl.ds(c * _G, _G), lo:lo + _HS]
            for j in range(_G):
                nsum_ref[j, lo:lo + _HS] = nsum_ref[j, lo:lo + _HS] + jnp.sum(
                    mc[j][:, None, :] * hc, axis=0)
            return carry

        jax.lax.fori_loop(0, g, _chunk, 0)

    md = adjE_ref[g]
    hnew = []
    for j in range(_G):
        oh_t = (iota32 == vt8[j:j + 1]).astype(jnp.float32)
        oh_p = (iota8 == vp8[j:j + 1]).astype(jnp.float32)
        oh = jnp.concatenate([oh_t, oh_p], axis=0)
        x = W40 @ oh + bhe + nsum_ref[j]
        for jp in range(j):
            x = x + md[j][jp:jp + 1] * hnew[jp]
        a = jnp.maximum(Wg1T_ref[...] @ x + bg1c_ref[...], 0.0)
        hnew.append(Wg2T_ref[...] @ a + bg2c_ref[...])

    Hg = jnp.concatenate(hnew, axis=0)
    h_ref[pl.ds(g * _G, _G)] = Hg.reshape(_G, _HID, bb)

    q_ref[...] += Wp1g_ref[0] @ Hg

    @pl.when(g == _NG - 1)
    def _readout():
        q = jnp.maximum(q_ref[...], 0.0)
        g_t = Wp2T_ref[...] @ q + bp2c_ref[...]
        sa = jnp.maximum(Ws1T_ref[...] @ vsT_ref[...] + bs1c_ref[...], 0.0)
        s_t = Ws2T_ref[...] @ sa + bs2c_ref[...]
        outT_ref[...] = WgpgT_ref[...] @ g_t + WgpsT_ref[...] @ s_t + bgpc_ref[...]


def kernel(v_types, v_paths, adj, v_sizes, type_table, path_table, Ws1, bs1,
           Ws2, bs2, Wh, bh, eps, Wg1, bg1, Wg2, bg2, Wp1, bp1, Wp2, bp2,
           Wgp, bgp):
    BB = 1024
    NB = _B // BB

    adjE = jnp.transpose(jnp.transpose(adj, (1, 2, 0)).reshape(_MAXN, 8, 8, _B),
                         (1, 0, 2, 3)).astype(jnp.float32)
    vtG = jnp.transpose(v_types, (1, 0)).reshape(_NG, _G, _B)
    vpG = jnp.transpose(v_paths, (1, 0)).reshape(_NG, _G, _B)
    vsT = jnp.transpose(v_sizes, (1, 0))
    Wp1g = jnp.transpose(Wp1.reshape(_NG, _G * _HID, 4 * _HID), (0, 2, 1))

    col = lambda x: x.reshape(-1, 1)
    rep = lambda shape: pl.BlockSpec(shape, lambda i, g: (0,) * len(shape))

    outT = pl.pallas_call(
        functools.partial(_digin_kernel, bb=BB),
        grid=(NB, _NG),
        in_specs=[
            pl.BlockSpec((8, _G, 8, BB), lambda i, g: (0, g, 0, i)),
            pl.BlockSpec((1, _G, BB), lambda i, g: (g, 0, i)),
            pl.BlockSpec((1, _G, BB), lambda i, g: (g, 0, i)),
            pl.BlockSpec((3 * _MAXN, BB), lambda i, g: (0, i)),
            rep((16, 32)),
            rep((16, 8)),
            rep((_HID, 16)),
            rep((_HID, 16)),
            rep((_HID, 1)),
            rep((1, 1)),
            rep((_HID, _HID)),
            rep((_HID, 1)),
            rep((_HID, _HID)),
            rep((_HID, 1)),
            pl.BlockSpec((1, 4 * _HID, _G * _HID),
                         lambda i, g: (g, 0, 0)),
            rep((4 * _HID, 1)),
            rep((_HID, 4 * _HID)),
            rep((_HID, 1)),
            rep((16, 3 * _MAXN)),
            rep((16, 1)),
            rep((8, 16)),
            rep((8, 1)),
            rep((_HID, _HID)),
            rep((_HID, 8)),
            rep((_HID, 1)),
        ],
        out_specs=pl.BlockSpec((_HID, BB), lambda i, g: (0, i)),
        out_shape=jax.ShapeDtypeStruct((_HID, _B), jnp.float32),
        scratch_shapes=[
            pltpu.VMEM((_MAXN, _HID, BB), jnp.float32),
            pltpu.VMEM((4 * _HID, BB), jnp.float32),
            pltpu.VMEM((_G, _HID, BB), jnp.float32),
        ],
    )(adjE, vtG, vpG, vsT,
      type_table.T, path_table.T, Wh[:16].T, Wh[16:].T, col(bh),
      eps.reshape(1, 1),
      Wg1.T, col(bg1), Wg2.T, col(bg2),
      Wp1g, col(bp1), Wp2.T, col(bp2),
      Ws1.T, col(bs1), Ws2.T, col(bs2),
      Wgp[:_HID].T, Wgp[_HID:].T, col(bgp))
    return outT.T


# speedup vs baseline: 8.2142x; 1.0069x over previous
import functools

import jax
import jax.numpy as jnp
from jax.experimental import pallas as pl
from jax.experimental.pallas import tpu as pltpu

_B = 4096
_MAXN = 64
_HID = 64
_G = 8
_NG = _MAXN // _G


def _digin_kernel(adjE_ref, vtG_ref, vpG_ref, vsT_ref, ttT_ref, ptT_ref,
                  WhtT_ref, WhpT_ref, bhc_ref, eps_ref,
                  Wg1T_ref, bg1c_ref, Wg2T_ref, bg2c_ref,
                  Wp1g_ref, bp1c_ref, Wp2T_ref, bp2c_ref,
                  Ws1T_ref, bs1c_ref, Ws2T_ref, bs2c_ref,
                  WgpgT_ref, WgpsT_ref, bgpc_ref,
                  outT_ref, h_ref, q_ref, nsum_ref, *, bb):
    g = pl.program_id(1)

    @pl.when(g == 0)
    def _init():
        q_ref[...] = jnp.broadcast_to(bp1c_ref[...], q_ref.shape)

    TtT = WhtT_ref[...] @ ttT_ref[...]
    PtT = WhpT_ref[...] @ ptT_ref[...]
    epsp1 = 1.0 + eps_ref[...]
    W40 = jnp.concatenate([TtT, PtT], axis=1) * epsp1
    bhe = bhc_ref[...] * epsp1
    vt8 = vtG_ref[0]
    vp8 = vpG_ref[0]
    iota32 = jax.lax.broadcasted_iota(jnp.int32, (32, bb), 0)
    iota8 = jax.lax.broadcasted_iota(jnp.int32, (8, bb), 0)

    nsum_ref[...] = jnp.zeros_like(nsum_ref)
    _HS = 16
    for s in range(_HID // _HS):
        lo = s * _HS

        def _chunk(c, carry, lo=lo):
            mc = adjE_ref[c].astype(jnp.float32)
            hc = h_ref[pl.ds(c * _G, _G), lo:lo + _HS]
            for j in range(_G):
                nsum_ref[j, lo:lo + _HS] = nsum_ref[j, lo:lo + _HS] + jnp.sum(
                    mc[j][:, None, :] * hc, axis=0)
            return carry

        jax.lax.fori_loop(0, g, _chunk, 0)

    md = adjE_ref[g].astype(jnp.float32)
    base = g * _G
    for j in range(_G):
        oh_t = (iota32 == vt8[j:j + 1]).astype(jnp.float32)
        oh_p = (iota8 == vp8[j:j + 1]).astype(jnp.float32)
        oh = jnp.concatenate([oh_t, oh_p], axis=0)
        x = W40 @ oh + bhe + nsum_ref[j]
        a = jnp.maximum(Wg1T_ref[...] @ x + bg1c_ref[...], 0.0)
        hj = Wg2T_ref[...] @ a + bg2c_ref[...]
        h_ref[base + j] = hj
        for jn in range(j + 1, _G):
            nsum_ref[jn] = nsum_ref[jn] + md[jn][j:j + 1] * hj

    Hg = h_ref[pl.ds(base, _G)].reshape(_G * _HID, bb)
    q_ref[...] += Wp1g_ref[0] @ Hg

    @pl.when(g == _NG - 1)
    def _readout():
        q = jnp.maximum(q_ref[...], 0.0)
        g_t = Wp2T_ref[...] @ q + bp2c_ref[...]
        sa = jnp.maximum(Ws1T_ref[...] @ vsT_ref[...] + bs1c_ref[...], 0.0)
        s_t = Ws2T_ref[...] @ sa + bs2c_ref[...]
        outT_ref[...] = WgpgT_ref[...] @ g_t + WgpsT_ref[...] @ s_t + bgpc_ref[...]


def kernel(v_types, v_paths, adj, v_sizes, type_table, path_table, Ws1, bs1,
           Ws2, bs2, Wh, bh, eps, Wg1, bg1, Wg2, bg2, Wp1, bp1, Wp2, bp2,
           Wgp, bgp):
    BB = 1024
    NB = _B // BB

    adjE = jnp.transpose(jnp.transpose(adj, (1, 2, 0)).reshape(_MAXN, 8, 8, _B),
                         (1, 0, 2, 3)).astype(jnp.int8)
    vtG = jnp.transpose(v_types, (1, 0)).reshape(_NG, _G, _B)
    vpG = jnp.transpose(v_paths, (1, 0)).reshape(_NG, _G, _B)
    vsT = jnp.transpose(v_sizes, (1, 0))
    Wp1g = jnp.transpose(Wp1.reshape(_NG, _G * _HID, 4 * _HID), (0, 2, 1))

    col = lambda x: x.reshape(-1, 1)
    rep = lambda shape: pl.BlockSpec(shape, lambda i, g: (0,) * len(shape))

    outT = pl.pallas_call(
        functools.partial(_digin_kernel, bb=BB),
        grid=(NB, _NG),
        in_specs=[
            pl.BlockSpec((8, _G, 8, BB), lambda i, g: (0, g, 0, i)),
            pl.BlockSpec((1, _G, BB), lambda i, g: (g, 0, i)),
            pl.BlockSpec((1, _G, BB), lambda i, g: (g, 0, i)),
            pl.BlockSpec((3 * _MAXN, BB), lambda i, g: (0, i)),
            rep((16, 32)),
            rep((16, 8)),
            rep((_HID, 16)),
            rep((_HID, 16)),
            rep((_HID, 1)),
            rep((1, 1)),
            rep((_HID, _HID)),
            rep((_HID, 1)),
            rep((_HID, _HID)),
            rep((_HID, 1)),
            pl.BlockSpec((1, 4 * _HID, _G * _HID),
                         lambda i, g: (g, 0, 0)),
            rep((4 * _HID, 1)),
            rep((_HID, 4 * _HID)),
            rep((_HID, 1)),
            rep((16, 3 * _MAXN)),
            rep((16, 1)),
            rep((8, 16)),
            rep((8, 1)),
            rep((_HID, _HID)),
            rep((_HID, 8)),
            rep((_HID, 1)),
        ],
        out_specs=pl.BlockSpec((_HID, BB), lambda i, g: (0, i)),
        out_shape=jax.ShapeDtypeStruct((_HID, _B), jnp.float32),
        scratch_shapes=[
            pltpu.VMEM((_MAXN, _HID, BB), jnp.float32),
            pltpu.VMEM((4 * _HID, BB), jnp.float32),
            pltpu.VMEM((_G, _HID, BB), jnp.float32),
        ],
    )(adjE, vtG, vpG, vsT,
      type_table.T, path_table.T, Wh[:16].T, Wh[16:].T, col(bh),
      eps.reshape(1, 1),
      Wg1.T, col(bg1), Wg2.T, col(bg2),
      Wp1g, col(bp1), Wp2.T, col(bp2),
      Ws1.T, col(bs1), Ws2.T, col(bs2),
      Wgp[:_HID].T, Wgp[_HID:].T, col(bgp))
    return outT.T


# speedup vs baseline: 9.5390x; 1.1613x over previous
import functools

import jax
import jax.numpy as jnp
from jax.experimental import pallas as pl
from jax.experimental.pallas import tpu as pltpu

_B = 4096
_MAXN = 64
_HID = 64
_G = 8
_NG = _MAXN // _G


def _mmf(a, b):
    return jax.lax.dot(a, b, preferred_element_type=jnp.float32)


def _digin_kernel(adjE_ref, vtG_ref, vpG_ref, vsT_ref, ttT_ref, ptT_ref,
                  WhtT_ref, WhpT_ref, bhc_ref, eps_ref,
                  Wg1T_ref, bg1c_ref, Wg2T_ref, bg2c_ref,
                  Wp1g_ref, bp1c_ref, Wp2T_ref, bp2c_ref,
                  Ws1T_ref, bs1c_ref, Ws2T_ref, bs2c_ref,
                  WgpgT_ref, WgpsT_ref, bgpc_ref,
                  outT_ref, h_ref, q_ref, nsum_ref, *, bb):
    g = pl.program_id(1)

    @pl.when(g == 0)
    def _init():
        q_ref[...] = jnp.broadcast_to(bp1c_ref[...], q_ref.shape)

    TtT = WhtT_ref[...] @ ttT_ref[...]
    PtT = WhpT_ref[...] @ ptT_ref[...]
    epsp1 = 1.0 + eps_ref[...]
    W40 = jnp.concatenate([TtT, PtT], axis=1) * epsp1
    bhe = bhc_ref[...] * epsp1
    vt8 = vtG_ref[0]
    vp8 = vpG_ref[0]
    iota32 = jax.lax.broadcasted_iota(jnp.int32, (32, bb), 0)
    iota8 = jax.lax.broadcasted_iota(jnp.int32, (8, bb), 0)

    nsum_ref[...] = jnp.zeros_like(nsum_ref)
    _HS = 16

    def _chunk(c, carry):
        mc = adjE_ref[c].astype(jnp.float32)
        for s in range(_HID // _HS):
            lo = s * _HS
            hc = h_ref[pl.ds(c * _G, _G), lo:lo + _HS]
            for j in range(_G):
                nsum_ref[j, lo:lo + _HS] = nsum_ref[j, lo:lo + _HS] + jnp.sum(
                    mc[j][:, None, :] * hc, axis=0)
        return carry

    jax.lax.fori_loop(0, g, _chunk, 0)

    md = adjE_ref[g].astype(jnp.float32)
    base = g * _G
    W40b = W40.astype(jnp.bfloat16)
    for j in range(_G):
        oh_t = (iota32 == vt8[j:j + 1]).astype(jnp.bfloat16)
        oh_p = (iota8 == vp8[j:j + 1]).astype(jnp.bfloat16)
        oh = jnp.concatenate([oh_t, oh_p], axis=0)
        x = _mmf(W40b, oh) + bhe + nsum_ref[j]
        a = jnp.maximum(_mmf(Wg1T_ref[...], x.astype(jnp.bfloat16))
                        + bg1c_ref[...], 0.0)
        hj = _mmf(Wg2T_ref[...], a.astype(jnp.bfloat16)) + bg2c_ref[...]
        h_ref[base + j] = hj
        for jn in range(j + 1, _G):
            nsum_ref[jn] = nsum_ref[jn] + md[jn][j:j + 1] * hj

    Hg = h_ref[pl.ds(base, _G)].reshape(_G * _HID, bb)
    q_ref[...] += _mmf(Wp1g_ref[0], Hg.astype(jnp.bfloat16))

    @pl.when(g == _NG - 1)
    def _readout():
        q = jnp.maximum(q_ref[...], 0.0)
        g_t = Wp2T_ref[...] @ q + bp2c_ref[...]
        sa = jnp.maximum(Ws1T_ref[...] @ vsT_ref[...] + bs1c_ref[...], 0.0)
        s_t = Ws2T_ref[...] @ sa + bs2c_ref[...]
        outT_ref[...] = WgpgT_ref[...] @ g_t + WgpsT_ref[...] @ s_t + bgpc_ref[...]


def kernel(v_types, v_paths, adj, v_sizes, type_table, path_table, Ws1, bs1,
           Ws2, bs2, Wh, bh, eps, Wg1, bg1, Wg2, bg2, Wp1, bp1, Wp2, bp2,
           Wgp, bgp):
    BB = 1024
    NB = _B // BB

    adjE = jnp.transpose(jnp.transpose(adj, (1, 2, 0)).reshape(_MAXN, 8, 8, _B),
                         (1, 0, 2, 3)).astype(jnp.int8)
    vtG = jnp.transpose(v_types, (1, 0)).reshape(_NG, _G, _B)
    vpG = jnp.transpose(v_paths, (1, 0)).reshape(_NG, _G, _B)
    vsT = jnp.transpose(v_sizes, (1, 0))
    Wp1g = jnp.transpose(Wp1.reshape(_NG, _G * _HID, 4 * _HID), (0, 2, 1))

    col = lambda x: x.reshape(-1, 1)
    rep = lambda shape: pl.BlockSpec(shape, lambda i, g: (0,) * len(shape))

    outT = pl.pallas_call(
        functools.partial(_digin_kernel, bb=BB),
        grid=(NB, _NG),
        in_specs=[
            pl.BlockSpec((8, _G, 8, BB), lambda i, g: (0, g, 0, i)),
            pl.BlockSpec((1, _G, BB), lambda i, g: (g, 0, i)),
            pl.BlockSpec((1, _G, BB), lambda i, g: (g, 0, i)),
            pl.BlockSpec((3 * _MAXN, BB), lambda i, g: (0, i)),
            rep((16, 32)),
            rep((16, 8)),
            rep((_HID, 16)),
            rep((_HID, 16)),
            rep((_HID, 1)),
            rep((1, 1)),
            rep((_HID, _HID)),
            rep((_HID, 1)),
            rep((_HID, _HID)),
            rep((_HID, 1)),
            pl.BlockSpec((1, 4 * _HID, _G * _HID),
                         lambda i, g: (g, 0, 0)),
            rep((4 * _HID, 1)),
            rep((_HID, 4 * _HID)),
            rep((_HID, 1)),
            rep((16, 3 * _MAXN)),
            rep((16, 1)),
            rep((8, 16)),
            rep((8, 1)),
            rep((_HID, _HID)),
            rep((_HID, 8)),
            rep((_HID, 1)),
        ],
        out_specs=pl.BlockSpec((_HID, BB), lambda i, g: (0, i)),
        out_shape=jax.ShapeDtypeStruct((_HID, _B), jnp.float32),
        scratch_shapes=[
            pltpu.VMEM((_MAXN, _HID, BB), jnp.float32),
            pltpu.VMEM((4 * _HID, BB), jnp.float32),
            pltpu.VMEM((_G, _HID, BB), jnp.float32),
        ],
    )(adjE, vtG, vpG, vsT,
      type_table.T, path_table.T, Wh[:16].T, Wh[16:].T, col(bh),
      eps.reshape(1, 1),
      Wg1.T.astype(jnp.bfloat16), col(bg1), Wg2.T.astype(jnp.bfloat16), col(bg2),
      Wp1g.astype(jnp.bfloat16), col(bp1), Wp2.T, col(bp2),
      Ws1.T, col(bs1), Ws2.T, col(bs2),
      Wgp[:_HID].T, Wgp[_HID:].T, col(bgp))
    return outT.T


# speedup vs baseline: 9.6468x; 1.0113x over previous
import functools

import jax
import jax.numpy as jnp
from jax.experimental import pallas as pl
from jax.experimental.pallas import tpu as pltpu

_B = 4096
_MAXN = 64
_HID = 64
_G = 8
_NG = _MAXN // _G


def _mmf(a, b):
    return jax.lax.dot(a, b, preferred_element_type=jnp.float32)


def _digin_kernel(adjE_ref, vtG_ref, vpG_ref, vsT_ref, ttT_ref, ptT_ref,
                  WhtT_ref, WhpT_ref, bhc_ref, eps_ref,
                  Wg1T_ref, bg1c_ref, Wg2T_ref, bg2c_ref,
                  Wp1g_ref, bp1c_ref, Wp2T_ref, bp2c_ref,
                  Ws1T_ref, bs1c_ref, Ws2T_ref, bs2c_ref,
                  WgpgT_ref, WgpsT_ref, bgpc_ref,
                  outT_ref, h_ref, q_ref, nsum_ref, *, bb):
    g = pl.program_id(1)

    @pl.when(g == 0)
    def _init():
        q_ref[...] = jnp.broadcast_to(bp1c_ref[...], q_ref.shape)

    TtT = WhtT_ref[...] @ ttT_ref[...]
    PtT = WhpT_ref[...] @ ptT_ref[...]
    epsp1 = 1.0 + eps_ref[...]
    W40 = jnp.concatenate([TtT, PtT], axis=1) * epsp1
    bhe = bhc_ref[...] * epsp1
    vt8 = vtG_ref[0]
    vp8 = vpG_ref[0]
    iota32 = jax.lax.broadcasted_iota(jnp.int32, (32, bb), 0)
    iota8 = jax.lax.broadcasted_iota(jnp.int32, (8, bb), 0)

    W40b = W40.astype(jnp.bfloat16)
    for j in range(_G):
        oh_t = (iota32 == vt8[j:j + 1]).astype(jnp.bfloat16)
        oh_p = (iota8 == vp8[j:j + 1]).astype(jnp.bfloat16)
        oh = jnp.concatenate([oh_t, oh_p], axis=0)
        nsum_ref[j] = _mmf(W40b, oh) + bhe

    _HS = 16

    def _chunk(c, carry):
        mc = adjE_ref[c].astype(jnp.float32)
        for s in range(_HID // _HS):
            lo = s * _HS
            hc = h_ref[pl.ds(c * _G, _G), lo:lo + _HS]
            for j in range(_G):
                nsum_ref[j, lo:lo + _HS] = nsum_ref[j, lo:lo + _HS] + jnp.sum(
                    mc[j][:, None, :] * hc, axis=0)
        return carry

    jax.lax.fori_loop(0, g, _chunk, 0)

    md = adjE_ref[g].astype(jnp.float32)
    base = g * _G
    for j in range(_G):
        x = nsum_ref[j]
        a = jnp.maximum(_mmf(Wg1T_ref[...], x.astype(jnp.bfloat16))
                        + bg1c_ref[...], 0.0)
        hj = _mmf(Wg2T_ref[...], a.astype(jnp.bfloat16)) + bg2c_ref[...]
        h_ref[base + j] = hj
        for jn in range(j + 1, _G):
            nsum_ref[jn] = nsum_ref[jn] + md[jn][j:j + 1] * hj

    Hg = h_ref[pl.ds(base, _G)].reshape(_G * _HID, bb)
    q_ref[...] += _mmf(Wp1g_ref[0], Hg.astype(jnp.bfloat16))

    @pl.when(g == _NG - 1)
    def _readout():
        q = jnp.maximum(q_ref[...], 0.0)
        g_t = Wp2T_ref[...] @ q + bp2c_ref[...]
        sa = jnp.maximum(Ws1T_ref[...] @ vsT_ref[...] + bs1c_ref[...], 0.0)
        s_t = Ws2T_ref[...] @ sa + bs2c_ref[...]
        outT_ref[...] = WgpgT_ref[...] @ g_t + WgpsT_ref[...] @ s_t + bgpc_ref[...]


def kernel(v_types, v_paths, adj, v_sizes, type_table, path_table, Ws1, bs1,
           Ws2, bs2, Wh, bh, eps, Wg1, bg1, Wg2, bg2, Wp1, bp1, Wp2, bp2,
           Wgp, bgp):
    BB = 1024
    NB = _B // BB

    adjE = jnp.transpose(jnp.transpose(adj, (1, 2, 0)).reshape(_MAXN, 8, 8, _B),
                         (1, 0, 2, 3)).astype(jnp.int8)
    vtG = jnp.transpose(v_types, (1, 0)).reshape(_NG, _G, _B)
    vpG = jnp.transpose(v_paths, (1, 0)).reshape(_NG, _G, _B)
    vsT = jnp.transpose(v_sizes, (1, 0))
    Wp1g = jnp.transpose(Wp1.reshape(_NG, _G * _HID, 4 * _HID), (0, 2, 1))

    col = lambda x: x.reshape(-1, 1)
    rep = lambda shape: pl.BlockSpec(shape, lambda i, g: (0,) * len(shape))

    outT = pl.pallas_call(
        functools.partial(_digin_kernel, bb=BB),
        grid=(NB, _NG),
        in_specs=[
            pl.BlockSpec((8, _G, 8, BB), lambda i, g: (0, g, 0, i)),
            pl.BlockSpec((1, _G, BB), lambda i, g: (g, 0, i)),
            pl.BlockSpec((1, _G, BB), lambda i, g: (g, 0, i)),
            pl.BlockSpec((3 * _MAXN, BB), lambda i, g: (0, i)),
            rep((16, 32)),
            rep((16, 8)),
            rep((_HID, 16)),
            rep((_HID, 16)),
            rep((_HID, 1)),
            rep((1, 1)),
            rep((_HID, _HID)),
            rep((_HID, 1)),
            rep((_HID, _HID)),
            rep((_HID, 1)),
            pl.BlockSpec((1, 4 * _HID, _G * _HID),
                         lambda i, g: (g, 0, 0)),
            rep((4 * _HID, 1)),
            rep((_HID, 4 * _HID)),
            rep((_HID, 1)),
            rep((16, 3 * _MAXN)),
            rep((16, 1)),
            rep((8, 16)),
            rep((8, 1)),
            rep((_HID, _HID)),
            rep((_HID, 8)),
            rep((_HID, 1)),
        ],
        out_specs=pl.BlockSpec((_HID, BB), lambda i, g: (0, i)),
        out_shape=jax.ShapeDtypeStruct((_HID, _B), jnp.float32),
        scratch_shapes=[
            pltpu.VMEM((_MAXN, _HID, BB), jnp.float32),
            pltpu.VMEM((4 * _HID, BB), jnp.float32),
            pltpu.VMEM((_G, _HID, BB), jnp.float32),
        ],
    )(adjE, vtG, vpG, vsT,
      type_table.T, path_table.T, Wh[:16].T, Wh[16:].T, col(bh),
      eps.reshape(1, 1),
      Wg1.T.astype(jnp.bfloat16), col(bg1), Wg2.T.astype(jnp.bfloat16), col(bg2),
      Wp1g.astype(jnp.bfloat16), col(bp1), Wp2.T, col(bp2),
      Ws1.T, col(bs1), Ws2.T, col(bs2),
      Wgp[:_HID].T, Wgp[_HID:].T, col(bgp))
    return outT.T


# speedup vs baseline: 10.0882x; 1.0458x over previous
import functools

import jax
import jax.numpy as jnp
from jax.experimental import pallas as pl
from jax.experimental.pallas import tpu as pltpu

_B = 4096
_MAXN = 64
_HID = 64
_G = 8
_NG = _MAXN // _G


def _mmf(a, b):
    return jax.lax.dot(a, b, preferred_element_type=jnp.float32)


def _digin_kernel(adjE_ref, vtG_ref, vpG_ref, vsT_ref, ttT_ref, ptT_ref,
                  WhtT_ref, WhpT_ref, bhc_ref, eps_ref,
                  Wg1T_ref, bg1c_ref, Wg2T_ref, bg2c_ref,
                  Wp1g_ref, bp1c_ref, Wp2T_ref, bp2c_ref,
                  Ws1T_ref, bs1c_ref, Ws2T_ref, bs2c_ref,
                  WgpgT_ref, WgpsT_ref, bgpc_ref,
                  outT_ref, h_ref, q_ref, nsum_ref, *, bb):
    g = pl.program_id(1)

    @pl.when(g == 0)
    def _init():
        q_ref[...] = jnp.broadcast_to(bp1c_ref[...], q_ref.shape)

    TtT = WhtT_ref[...] @ ttT_ref[...]
    PtT = WhpT_ref[...] @ ptT_ref[...]
    epsp1 = 1.0 + eps_ref[...]
    W40 = jnp.concatenate([TtT, PtT], axis=1) * epsp1
    bhe = bhc_ref[...] * epsp1
    vt8 = vtG_ref[0]
    vp8 = vpG_ref[0]
    iota32 = jax.lax.broadcasted_iota(jnp.int32, (32, bb), 0)
    iota8 = jax.lax.broadcasted_iota(jnp.int32, (8, bb), 0)

    W40b = W40.astype(jnp.bfloat16)
    for j in range(_G):
        oh_t = (iota32 == vt8[j:j + 1]).astype(jnp.bfloat16)
        oh_p = (iota8 == vp8[j:j + 1]).astype(jnp.bfloat16)
        oh = jnp.concatenate([oh_t, oh_p], axis=0)
        nsum_ref[j] = _mmf(W40b, oh) + bhe

    _HS = 32

    def _chunk(c, carry):
        mc = adjE_ref[c].astype(jnp.float32)
        for s in range(_HID // _HS):
            lo = s * _HS
            hc = h_ref[pl.ds(c * _G, _G), lo:lo + _HS]
            for j in range(_G):
                nsum_ref[j, lo:lo + _HS] = nsum_ref[j, lo:lo + _HS] + jnp.sum(
                    mc[j][:, None, :] * hc, axis=0)
        return carry

    jax.lax.fori_loop(0, g, _chunk, 0)

    md = adjE_ref[g].astype(jnp.float32)
    base = g * _G
    for j in range(_G):
        x = nsum_ref[j]
        a = jnp.maximum(_mmf(Wg1T_ref[...], x.astype(jnp.bfloat16))
                        + bg1c_ref[...], 0.0)
        hj = _mmf(Wg2T_ref[...], a.astype(jnp.bfloat16)) + bg2c_ref[...]
        h_ref[base + j] = hj
        for jn in range(j + 1, _G):
            nsum_ref[jn] = nsum_ref[jn] + md[jn][j:j + 1] * hj

    Hg = h_ref[pl.ds(base, _G)].reshape(_G * _HID, bb)
    q_ref[...] += _mmf(Wp1g_ref[0], Hg.astype(jnp.bfloat16))

    @pl.when(g == _NG - 1)
    def _readout():
        q = jnp.maximum(q_ref[...], 0.0)
        g_t = Wp2T_ref[...] @ q + bp2c_ref[...]
        sa = jnp.maximum(Ws1T_ref[...] @ vsT_ref[...] + bs1c_ref[...], 0.0)
        s_t = Ws2T_ref[...] @ sa + bs2c_ref[...]
        outT_ref[...] = WgpgT_ref[...] @ g_t + WgpsT_ref[...] @ s_t + bgpc_ref[...]


def kernel(v_types, v_paths, adj, v_sizes, type_table, path_table, Ws1, bs1,
           Ws2, bs2, Wh, bh, eps, Wg1, bg1, Wg2, bg2, Wp1, bp1, Wp2, bp2,
           Wgp, bgp):
    BB = 1024
    NB = _B // BB

    adjE = jnp.transpose(jnp.transpose(adj, (1, 2, 0)).reshape(_MAXN, 8, 8, _B),
                         (1, 0, 2, 3)).astype(jnp.int8)
    vtG = jnp.transpose(v_types, (1, 0)).reshape(_NG, _G, _B)
    vpG = jnp.transpose(v_paths, (1, 0)).reshape(_NG, _G, _B)
    vsT = jnp.transpose(v_sizes, (1, 0))
    Wp1g = jnp.transpose(Wp1.reshape(_NG, _G * _HID, 4 * _HID), (0, 2, 1))

    col = lambda x: x.reshape(-1, 1)
    rep = lambda shape: pl.BlockSpec(shape, lambda i, g: (0,) * len(shape))

    outT = pl.pallas_call(
        functools.partial(_digin_kernel, bb=BB),
        grid=(NB, _NG),
        in_specs=[
            pl.BlockSpec((8, _G, 8, BB), lambda i, g: (0, g, 0, i)),
            pl.BlockSpec((1, _G, BB), lambda i, g: (g, 0, i)),
            pl.BlockSpec((1, _G, BB), lambda i, g: (g, 0, i)),
            pl.BlockSpec((3 * _MAXN, BB), lambda i, g: (0, i)),
            rep((16, 32)),
            rep((16, 8)),
            rep((_HID, 16)),
            rep((_HID, 16)),
            rep((_HID, 1)),
            rep((1, 1)),
            rep((_HID, _HID)),
            rep((_HID, 1)),
            rep((_HID, _HID)),
            rep((_HID, 1)),
            pl.BlockSpec((1, 4 * _HID, _G * _HID),
                         lambda i, g: (g, 0, 0)),
            rep((4 * _HID, 1)),
            rep((_HID, 4 * _HID)),
            rep((_HID, 1)),
            rep((16, 3 * _MAXN)),
            rep((16, 1)),
            rep((8, 16)),
            rep((8, 1)),
            rep((_HID, _HID)),
            rep((_HID, 8)),
            rep((_HID, 1)),
        ],
        out_specs=pl.BlockSpec((_HID, BB), lambda i, g: (0, i)),
        out_shape=jax.ShapeDtypeStruct((_HID, _B), jnp.float32),
        scratch_shapes=[
            pltpu.VMEM((_MAXN, _HID, BB), jnp.float32),
            pltpu.VMEM((4 * _HID, BB), jnp.float32),
            pltpu.VMEM((_G, _HID, BB), jnp.float32),
        ],
    )(adjE, vtG, vpG, vsT,
      type_table.T, path_table.T, Wh[:16].T, Wh[16:].T, col(bh),
      eps.reshape(1, 1),
      Wg1.T.astype(jnp.bfloat16), col(bg1), Wg2.T.astype(jnp.bfloat16), col(bg2),
      Wp1g.astype(jnp.bfloat16), col(bp1), Wp2.T, col(bp2),
      Ws1.T, col(bs1), Ws2.T, col(bs2),
      Wgp[:_HID].T, Wgp[_HID:].T, col(bgp))
    return outT.T


# speedup vs baseline: 11.0781x; 1.0981x over previous
import functools

import jax
import jax.numpy as jnp
from jax.experimental import pallas as pl
from jax.experimental.pallas import tpu as pltpu

_B = 4096
_MAXN = 64
_HID = 64
_G = 8
_NG = _MAXN // _G


def _mmf(a, b):
    return jax.lax.dot(a, b, preferred_element_type=jnp.float32)


def _digin_kernel(adjE_ref, vtG_ref, vpG_ref, vsT_ref, ttT_ref, ptT_ref,
                  WhtT_ref, WhpT_ref, bhc_ref, eps_ref,
                  Wg1T_ref, bg1c_ref, Wg2T_ref, bg2c_ref,
                  Wp1g_ref, bp1c_ref, Wp2T_ref, bp2c_ref,
                  Ws1T_ref, bs1c_ref, Ws2T_ref, bs2c_ref,
                  WgpgT_ref, WgpsT_ref, bgpc_ref,
                  outT_ref, h_ref, q_ref, nsum_ref, *, bb):
    g = pl.program_id(1)

    @pl.when(g == 0)
    def _init():
        q_ref[...] = jnp.broadcast_to(bp1c_ref[...], q_ref.shape)

    TtT = WhtT_ref[...] @ ttT_ref[...]
    PtT = WhpT_ref[...] @ ptT_ref[...]
    epsp1 = 1.0 + eps_ref[...]
    W40 = jnp.concatenate([TtT, PtT], axis=1) * epsp1
    bhe = bhc_ref[...] * epsp1
    vt8 = vtG_ref[0]
    vp8 = vpG_ref[0]
    iota32 = jax.lax.broadcasted_iota(jnp.int32, (32, bb), 0)
    iota8 = jax.lax.broadcasted_iota(jnp.int32, (8, bb), 0)

    W40b = W40.astype(jnp.bfloat16)
    for j in range(_G):
        oh_t = (iota32 == vt8[j:j + 1]).astype(jnp.bfloat16)
        oh_p = (iota8 == vp8[j:j + 1]).astype(jnp.bfloat16)
        oh = jnp.concatenate([oh_t, oh_p], axis=0)
        nsum_ref[j] = _mmf(W40b, oh) + bhe

    _HS = 16

    def _chunk(c, carry):
        mc = adjE_ref[c].astype(jnp.float32)
        for s in range(_HID // _HS):
            lo = s * _HS
            hc = h_ref[pl.ds(c * _G, _G), lo:lo + _HS]
            for j in range(_G):
                nsum_ref[j, lo:lo + _HS] = nsum_ref[j, lo:lo + _HS] + jnp.sum(
                    mc[j][:, None, :] * hc, axis=0)
        return carry

    jax.lax.fori_loop(0, g, _chunk, 0)

    md = adjE_ref[g].astype(jnp.float32)
    base = g * _G
    for j in range(_G):
        x = nsum_ref[j]
        a = jnp.maximum(_mmf(Wg1T_ref[...], x.astype(jnp.bfloat16))
                        + bg1c_ref[...], 0.0)
        hj = _mmf(Wg2T_ref[...], a.astype(jnp.bfloat16)) + bg2c_ref[...]
        h_ref[base + j] = hj
        for jn in range(j + 1, _G):
            nsum_ref[jn] = nsum_ref[jn] + md[jn][j:j + 1] * hj

    Hg = h_ref[pl.ds(base, _G)].reshape(_G * _HID, bb)
    q_ref[...] += _mmf(Wp1g_ref[0], Hg.astype(jnp.bfloat16))

    @pl.when(g == _NG - 1)
    def _readout():
        q = jnp.maximum(q_ref[...], 0.0)
        g_t = Wp2T_ref[...] @ q + bp2c_ref[...]
        sa = jnp.maximum(Ws1T_ref[...] @ vsT_ref[...] + bs1c_ref[...], 0.0)
        s_t = Ws2T_ref[...] @ sa + bs2c_ref[...]
        outT_ref[...] = WgpgT_ref[...] @ g_t + WgpsT_ref[...] @ s_t + bgpc_ref[...]


def kernel(v_types, v_paths, adj, v_sizes, type_table, path_table, Ws1, bs1,
           Ws2, bs2, Wh, bh, eps, Wg1, bg1, Wg2, bg2, Wp1, bp1, Wp2, bp2,
           Wgp, bgp):
    BB = 2048
    NB = _B // BB

    adjE = jnp.transpose(jnp.transpose(adj, (1, 2, 0)).reshape(_MAXN, 8, 8, _B),
                         (1, 0, 2, 3)).astype(jnp.int8)
    vtG = jnp.transpose(v_types, (1, 0)).reshape(_NG, _G, _B)
    vpG = jnp.transpose(v_paths, (1, 0)).reshape(_NG, _G, _B)
    vsT = jnp.transpose(v_sizes, (1, 0))
    Wp1g = jnp.transpose(Wp1.reshape(_NG, _G * _HID, 4 * _HID), (0, 2, 1))

    col = lambda x: x.reshape(-1, 1)
    rep = lambda shape: pl.BlockSpec(shape, lambda i, g: (0,) * len(shape))

    outT = pl.pallas_call(
        functools.partial(_digin_kernel, bb=BB),
        grid=(NB, _NG),
        in_specs=[
            pl.BlockSpec((8, _G, 8, BB), lambda i, g: (0, g, 0, i)),
            pl.BlockSpec((1, _G, BB), lambda i, g: (g, 0, i)),
            pl.BlockSpec((1, _G, BB), lambda i, g: (g, 0, i)),
            pl.BlockSpec((3 * _MAXN, BB), lambda i, g: (0, i)),
            rep((16, 32)),
            rep((16, 8)),
            rep((_HID, 16)),
            rep((_HID, 16)),
            rep((_HID, 1)),
            rep((1, 1)),
            rep((_HID, _HID)),
            rep((_HID, 1)),
            rep((_HID, _HID)),
            rep((_HID, 1)),
            pl.BlockSpec((1, 4 * _HID, _G * _HID),
                         lambda i, g: (g, 0, 0)),
            rep((4 * _HID, 1)),
            rep((_HID, 4 * _HID)),
            rep((_HID, 1)),
            rep((16, 3 * _MAXN)),
            rep((16, 1)),
            rep((8, 16)),
            rep((8, 1)),
            rep((_HID, _HID)),
            rep((_HID, 8)),
            rep((_HID, 1)),
        ],
        out_specs=pl.BlockSpec((_HID, BB), lambda i, g: (0, i)),
        out_shape=jax.ShapeDtypeStruct((_HID, _B), jnp.float32),
        scratch_shapes=[
            pltpu.VMEM((_MAXN, _HID, BB), jnp.float32),
            pltpu.VMEM((4 * _HID, BB), jnp.float32),
            pltpu.VMEM((_G, _HID, BB), jnp.float32),
        ],
    )(adjE, vtG, vpG, vsT,
      type_table.T, path_table.T, Wh[:16].T, Wh[16:].T, col(bh),
      eps.reshape(1, 1),
      Wg1.T.astype(jnp.bfloat16), col(bg1), Wg2.T.astype(jnp.bfloat16), col(bg2),
      Wp1g.astype(jnp.bfloat16), col(bp1), Wp2.T, col(bp2),
      Ws1.T, col(bs1), Ws2.T, col(bs2),
      Wgp[:_HID].T, Wgp[_HID:].T, col(bgp))
    return outT.T
